# baseline (speedup 1.0000x reference)
import functools

import jax
import jax.numpy as jnp
from jax import lax
from jax.experimental import pallas as pl
from jax.experimental.pallas import tpu as pltpu
from jax.experimental.pallas import tpu_sc as plsc

N = 320000
C = 128
K = 4
P = 16
CAT = 32

NC = 2
NS = 16
NW = NC * NS
ROWS_W = N // NW
B = 80
NCH = ROWS_W // B
LANES = 16


CW = C // 2


def _pack_row(z):
    zb = z.astype(jnp.bfloat16)
    hi = lax.bitcast_convert_type(zb[:, :CW], jnp.uint16).astype(jnp.uint32)
    lo = lax.bitcast_convert_type(zb[:, CW:], jnp.uint16).astype(jnp.uint32)
    return (hi << 16) | lo


def _unpack_row(v):
    hi = lax.bitcast_convert_type(
        lax.shift_right_logical(v, jnp.uint32(16)).astype(jnp.uint16),
        jnp.bfloat16)
    lo = lax.bitcast_convert_type(
        (v & jnp.uint32(0xFFFF)).astype(jnp.uint16), jnp.bfloat16)
    return hi, lo


def _proj_body(first, x_ref, ws_ref, wn_ref, b_ref, zself_ref, zn_ref):
    if first:
        xa = x_ref[...].astype(jnp.bfloat16)
        dots = [jnp.dot(xa, ws_ref[...], preferred_element_type=jnp.float32)]
        dots += [jnp.dot(xa, wn_ref[k], preferred_element_type=jnp.float32)
                 for k in range(K)]
    else:
        xa, xb = _unpack_row(x_ref[...])
        dots = [
            jnp.dot(xa, ws_ref[0], preferred_element_type=jnp.float32)
            + jnp.dot(xb, ws_ref[1], preferred_element_type=jnp.float32)
        ]
        dots += [
            jnp.dot(xa, wn_ref[k, 0], preferred_element_type=jnp.float32)
            + jnp.dot(xb, wn_ref[k, 1], preferred_element_type=jnp.float32)
            for k in range(K)
        ]
    zself_ref[...] = _pack_row(dots[0] + b_ref[...])
    for k in range(K):
        zn_ref[k] = dots[k + 1]


@functools.lru_cache(maxsize=None)
def _make_project(first, n, bn):
    grid = n // bn
    if first:
        x_spec = pl.BlockSpec((bn, C), lambda i: (i, 0))
        ws_spec = pl.BlockSpec((C, C), lambda i: (0, 0))
        wn_spec = pl.BlockSpec((K, C, C), lambda i: (0, 0, 0))
    else:
        x_spec = pl.BlockSpec((bn, CW), lambda i: (i, 0))
        ws_spec = pl.BlockSpec((2, CW, C), lambda i: (0, 0, 0))
        wn_spec = pl.BlockSpec((K, 2, CW, C), lambda i: (0, 0, 0, 0))
    return pl.pallas_call(
        functools.partial(_proj_body, first),
        grid=(grid,),
        in_specs=[
            x_spec,
            ws_spec,
            wn_spec,
            pl.BlockSpec((1, C), lambda i: (0, 0)),
        ],
        out_specs=[
            pl.BlockSpec((bn, CW), lambda i: (i, 0)),
            pl.BlockSpec((K, bn, C), lambda i: (0, i, 0)),
        ],
        out_shape=[
            jax.ShapeDtypeStruct((n, CW), jnp.uint32),
            jax.ShapeDtypeStruct((K, n, C), jnp.float32),
        ],
    )


_M_HI = jnp.uint32(0xFFFF0000)
_R_HALF = jnp.uint32(0x8000)
_S16 = jnp.uint32(16)


def _sc_combine_rows(acc_v, gbuf_v):
    def f32hi(u):
        return lax.bitcast_convert_type(u & _M_HI, jnp.float32)

    def f32lo(u):
        return lax.bitcast_convert_type(lax.shift_left(u, _S16), jnp.float32)

    def row(r, carry):
        for cc in range(CW // LANES):
            slp = pl.ds(cc * LANES, LANES)
            slo = pl.ds(CW + cc * LANES, LANES)
            u = acc_v[r, slp]
            ga = ((gbuf_v[0, r, slp] + gbuf_v[1, r, slp])
                  + (gbuf_v[2, r, slp] + gbuf_v[3, r, slp]))
            gb = ((gbuf_v[0, r, slo] + gbuf_v[1, r, slo])
                  + (gbuf_v[2, r, slo] + gbuf_v[3, r, slo]))
            a = jnp.maximum(f32hi(u) + ga, 0.0)
            b = jnp.maximum(f32lo(u) + gb, 0.0)
            ua = (lax.bitcast_convert_type(a, jnp.uint32) + _R_HALF) & _M_HI
            ub = lax.shift_right_logical(
                lax.bitcast_convert_type(b, jnp.uint32) + _R_HALF, _S16)
            acc_v[r, slp] = ua | ub
        return carry

    lax.fori_loop(0, B, row, 0)


def _sc_body(zself_hbm, znf_hbm, idx_hbm, out_hbm,
             idx0, idx1, acc0, acc1, gb0, gb1, sg0, sg1, si0, si1):
    wid = lax.axis_index("s") * NC + lax.axis_index("c")
    base_c = wid * NCH
    idxb, accb, gbb = (idx0, idx1), (acc0, acc1), (gb0, gb1)
    sgb, sib = (sg0, sg1), (si0, si1)

    def issue(ci, b):
        for k in range(K):
            pltpu.async_copy(znf_hbm.at[idxb[b].at[k]], gbb[b].at[k], sgb[b])
        pltpu.async_copy(
            zself_hbm.at[pl.ds((base_c + ci) * B, B)], accb[b], sgb[b])

    def drain(b):
        for k in range(K):
            pltpu.make_async_copy(
                znf_hbm.at[idxb[b].at[k]], gbb[b].at[k], sgb[b]).wait()
        pltpu.make_async_copy(
            zself_hbm.at[pl.ds(0, B)], accb[b], sgb[b]).wait()

    def issue_idx(ci, b):
        pltpu.async_copy(idx_hbm.at[base_c + ci], idxb[b], sib[b])

    def drain_idx(b):
        pltpu.make_async_copy(idx_hbm.at[0], idxb[b], sib[b]).wait()

    def process(ci, b):
        drain(b)

        @pl.when(ci + 1 < NCH)
        def _start_next():
            drain_idx(1 - b)
            issue(ci + 1, 1 - b)

        @pl.when(ci + 2 < NCH)
        def _prefetch_idx():
            issue_idx(ci + 2, b)

        _sc_combine_rows(accb[b], gbb[b])
        pltpu.sync_copy(accb[b], out_hbm.at[pl.ds((base_c + ci) * B, B)])

    pltpu.sync_copy(idx_hbm.at[base_c], idxb[0])
    issue(0, 0)
    issue_idx(1, 1)

    def pair(i, carry):
        ci = i * 2
        process(ci, 0)

        @pl.when(ci + 1 < NCH)
        def _odd():
            process(ci + 1, 1)

        return carry

    lax.fori_loop(0, (NCH + 1) // 2, pair, 0)


@functools.lru_cache(maxsize=None)
def _make_sc_conv():
    mesh = plsc.VectorSubcoreMesh(core_axis_name="c", subcore_axis_name="s",
                                  num_cores=NC, num_subcores=NS)
    scratch = [
        pltpu.VMEM((K, B), jnp.int32),
        pltpu.VMEM((K, B), jnp.int32),
        pltpu.VMEM((B, CW), jnp.uint32),
        pltpu.VMEM((B, CW), jnp.uint32),
        pltpu.VMEM((K, B, C), jnp.float32),
        pltpu.VMEM((K, B, C), jnp.float32),
        pltpu.SemaphoreType.DMA,
        pltpu.SemaphoreType.DMA,
        pltpu.SemaphoreType.DMA,
        pltpu.SemaphoreType.DMA,
    ]
    return pl.kernel(
        _sc_body,
        out_type=jax.ShapeDtypeStruct((N, CW), jnp.uint32),
        mesh=mesh,
        scratch_types=scratch,
    )


BN_POOL = 4000
NBLK = N // BN_POOL
BLK_PER_BIN = (N // P) // BN_POOL


def _pool_fc_body(h_ref, wf3_ref, bfc_ref, out_ref, pooled_ref):
    i = pl.program_id(0)
    r = i // BLK_PER_BIN
    ha, hb = _unpack_row(h_ref[...])
    s = jnp.concatenate(
        [jnp.sum(ha.astype(jnp.float32), axis=0, keepdims=True),
         jnp.sum(hb.astype(jnp.float32), axis=0, keepdims=True)], axis=1)

    @pl.when(i % BLK_PER_BIN == 0)
    def _init():
        pooled_ref[pl.ds(r, 1), :] = s

    @pl.when(i % BLK_PER_BIN != 0)
    def _acc():
        pooled_ref[pl.ds(r, 1), :] = pooled_ref[pl.ds(r, 1), :] + s

    @pl.when(i == NBLK - 1)
    def _fc():
        acc = bfc_ref[...]
        for p in range(P):
            acc = acc + jnp.dot(pooled_ref[p:p + 1, :], wf3_ref[p],
                                preferred_element_type=jnp.float32)
        out_ref[...] = acc


@functools.lru_cache(maxsize=None)
def _make_pool_fc():
    return pl.pallas_call(
        _pool_fc_body,
        grid=(NBLK,),
        in_specs=[
            pl.BlockSpec((BN_POOL, CW), lambda i: (i, 0)),
            pl.BlockSpec((P, C, CAT), lambda i: (0, 0, 0)),
            pl.BlockSpec((1, CAT), lambda i: (0, 0)),
        ],
        out_specs=pl.BlockSpec((1, CAT), lambda i: (0, 0)),
        out_shape=jax.ShapeDtypeStruct((1, CAT), jnp.float32),
        scratch_shapes=[pltpu.VMEM((P, C), jnp.float32)],
    )


def _split_weights(W, first):
    Wr = W.reshape(C, K + 1, C)
    ws = Wr[:, 0, :].T
    wn = Wr[:, 1:, :].transpose(1, 2, 0)
    ws = ws.astype(jnp.bfloat16)
    wn = wn.astype(jnp.bfloat16)
    if not first:
        ws = ws.reshape(2, CW, C)
        wn = wn.reshape(K, 2, CW, C)
    return ws, wn


def kernel(x, half_edges, W0, b0, W1, b1, W2, b2, Wfc, bfc):
    he = half_edges.astype(jnp.int32)
    idx_full = he.T + (jnp.arange(K, dtype=jnp.int32) * N)[:, None]
    idx_tiled = idx_full.reshape(K, NW * NCH, B).transpose(1, 0, 2)

    sc_conv = _make_sc_conv()

    h = x
    for li, (W, b) in enumerate(((W0, b0), (W1, b1), (W2, b2))):
        first = li == 0
        ws, wn = _split_weights(W, first)
        zself, zn = _make_project(first, N, 3200)(h, ws, wn, b.reshape(1, C))
        znf = zn.reshape(K * N, C)
        h = sc_conv(zself, znf, idx_tiled)

    wf3 = Wfc.reshape(CAT, P, C).transpose(1, 2, 0) * (1.0 / (N // P))
    out = _make_pool_fc()(h, wf3, bfc.reshape(1, CAT))
    return out.reshape(CAT)


# speedup vs baseline: 1.0047x; 1.0047x over previous
import functools

import jax
import jax.numpy as jnp
from jax import lax
from jax.experimental import pallas as pl
from jax.experimental.pallas import tpu as pltpu
from jax.experimental.pallas import tpu_sc as plsc

N = 320000
C = 128
K = 4
P = 16
CAT = 32

NC = 2
NS = 16
NW = NC * NS
ROWS_W = N // NW
B = 80
NCH = ROWS_W // B
LANES = 16


CW = C // 2


def _pack_row(z):
    zb = z.astype(jnp.bfloat16)
    hi = lax.bitcast_convert_type(zb[:, :CW], jnp.uint16).astype(jnp.uint32)
    lo = lax.bitcast_convert_type(zb[:, CW:], jnp.uint16).astype(jnp.uint32)
    return (hi << 16) | lo


def _unpack_row(v):
    hi = lax.bitcast_convert_type(
        lax.shift_right_logical(v, jnp.uint32(16)).astype(jnp.uint16),
        jnp.bfloat16)
    lo = lax.bitcast_convert_type(
        (v & jnp.uint32(0xFFFF)).astype(jnp.uint16), jnp.bfloat16)
    return hi, lo


def _proj_body(first, x_ref, ws_ref, wn_ref, b_ref, zself_ref, zn_ref):
    if first:
        xa = x_ref[...].astype(jnp.bfloat16)
    else:
        hi, lo = _unpack_row(x_ref[...])
        xa = jnp.concatenate([hi, lo], axis=1)
    dots = [jnp.dot(xa, ws_ref[...], preferred_element_type=jnp.float32)]
    dots += [jnp.dot(xa, wn_ref[k], preferred_element_type=jnp.float32)
             for k in range(K)]
    zself_ref[...] = _pack_row(dots[0] + b_ref[...])
    for k in range(K):
        zn_ref[k] = dots[k + 1]


@functools.lru_cache(maxsize=None)
def _make_project(first, n, bn):
    grid = n // bn
    if first:
        x_spec = pl.BlockSpec((bn, C), lambda i: (i, 0))
    else:
        x_spec = pl.BlockSpec((bn, CW), lambda i: (i, 0))
    ws_spec = pl.BlockSpec((C, C), lambda i: (0, 0))
    wn_spec = pl.BlockSpec((K, C, C), lambda i: (0, 0, 0))
    return pl.pallas_call(
        functools.partial(_proj_body, first),
        grid=(grid,),
        in_specs=[
            x_spec,
            ws_spec,
            wn_spec,
            pl.BlockSpec((1, C), lambda i: (0, 0)),
        ],
        out_specs=[
            pl.BlockSpec((bn, CW), lambda i: (i, 0)),
            pl.BlockSpec((K, bn, C), lambda i: (0, i, 0)),
        ],
        out_shape=[
            jax.ShapeDtypeStruct((n, CW), jnp.uint32),
            jax.ShapeDtypeStruct((K, n, C), jnp.float32),
        ],
    )


_M_HI = jnp.uint32(0xFFFF0000)
_R_HALF = jnp.uint32(0x8000)
_S16 = jnp.uint32(16)


def _sc_combine_rows(acc_v, gbuf_v):
    def f32hi(u):
        return lax.bitcast_convert_type(u & _M_HI, jnp.float32)

    def f32lo(u):
        return lax.bitcast_convert_type(lax.shift_left(u, _S16), jnp.float32)

    def row(r, carry):
        for cc in range(CW // LANES):
            slp = pl.ds(cc * LANES, LANES)
            slo = pl.ds(CW + cc * LANES, LANES)
            u = acc_v[r, slp]
            ga = ((gbuf_v[0, r, slp] + gbuf_v[1, r, slp])
                  + (gbuf_v[2, r, slp] + gbuf_v[3, r, slp]))
            gb = ((gbuf_v[0, r, slo] + gbuf_v[1, r, slo])
                  + (gbuf_v[2, r, slo] + gbuf_v[3, r, slo]))
            a = jnp.maximum(f32hi(u) + ga, 0.0)
            b = jnp.maximum(f32lo(u) + gb, 0.0)
            ua = (lax.bitcast_convert_type(a, jnp.uint32) + _R_HALF) & _M_HI
            ub = lax.shift_right_logical(
                lax.bitcast_convert_type(b, jnp.uint32) + _R_HALF, _S16)
            acc_v[r, slp] = ua | ub
        return carry

    lax.fori_loop(0, B, row, 0)


def _sc_body(zself_hbm, znf_hbm, idx_hbm, out_hbm,
             idx0, idx1, acc0, acc1, gb0, gb1, sg0, sg1, si0, si1):
    wid = lax.axis_index("s") * NC + lax.axis_index("c")
    base_c = wid * NCH
    idxb, accb, gbb = (idx0, idx1), (acc0, acc1), (gb0, gb1)
    sgb, sib = (sg0, sg1), (si0, si1)

    def issue(ci, b):
        for k in range(K):
            pltpu.async_copy(znf_hbm.at[idxb[b].at[k]], gbb[b].at[k], sgb[b])
        pltpu.async_copy(
            zself_hbm.at[pl.ds((base_c + ci) * B, B)], accb[b], sgb[b])

    def drain(b):
        for k in range(K):
            pltpu.make_async_copy(
                znf_hbm.at[idxb[b].at[k]], gbb[b].at[k], sgb[b]).wait()
        pltpu.make_async_copy(
            zself_hbm.at[pl.ds(0, B)], accb[b], sgb[b]).wait()

    def issue_idx(ci, b):
        pltpu.async_copy(idx_hbm.at[base_c + ci], idxb[b], sib[b])

    def drain_idx(b):
        pltpu.make_async_copy(idx_hbm.at[0], idxb[b], sib[b]).wait()

    def process(ci, b):
        drain(b)

        @pl.when(ci + 1 < NCH)
        def _start_next():
            drain_idx(1 - b)
            issue(ci + 1, 1 - b)

        @pl.when(ci + 2 < NCH)
        def _prefetch_idx():
            issue_idx(ci + 2, b)

        _sc_combine_rows(accb[b], gbb[b])
        pltpu.sync_copy(accb[b], out_hbm.at[pl.ds((base_c + ci) * B, B)])

    pltpu.sync_copy(idx_hbm.at[base_c], idxb[0])
    issue(0, 0)
    issue_idx(1, 1)

    def pair(i, carry):
        ci = i * 2
        process(ci, 0)

        @pl.when(ci + 1 < NCH)
        def _odd():
            process(ci + 1, 1)

        return carry

    lax.fori_loop(0, (NCH + 1) // 2, pair, 0)


@functools.lru_cache(maxsize=None)
def _make_sc_conv():
    mesh = plsc.VectorSubcoreMesh(core_axis_name="c", subcore_axis_name="s",
                                  num_cores=NC, num_subcores=NS)
    scratch = [
        pltpu.VMEM((K, B), jnp.int32),
        pltpu.VMEM((K, B), jnp.int32),
        pltpu.VMEM((B, CW), jnp.uint32),
        pltpu.VMEM((B, CW), jnp.uint32),
        pltpu.VMEM((K, B, C), jnp.float32),
        pltpu.VMEM((K, B, C), jnp.float32),
        pltpu.SemaphoreType.DMA,
        pltpu.SemaphoreType.DMA,
        pltpu.SemaphoreType.DMA,
        pltpu.SemaphoreType.DMA,
    ]
    return pl.kernel(
        _sc_body,
        out_type=jax.ShapeDtypeStruct((N, CW), jnp.uint32),
        mesh=mesh,
        scratch_types=scratch,
    )


BN_POOL = 4000
NBLK = N // BN_POOL
BLK_PER_BIN = (N // P) // BN_POOL


def _pool_fc_body(h_ref, wf3_ref, bfc_ref, out_ref, pooled_ref):
    i = pl.program_id(0)
    r = i // BLK_PER_BIN
    ha, hb = _unpack_row(h_ref[...])
    s = jnp.concatenate(
        [jnp.sum(ha.astype(jnp.float32), axis=0, keepdims=True),
         jnp.sum(hb.astype(jnp.float32), axis=0, keepdims=True)], axis=1)

    @pl.when(i % BLK_PER_BIN == 0)
    def _init():
        pooled_ref[pl.ds(r, 1), :] = s

    @pl.when(i % BLK_PER_BIN != 0)
    def _acc():
        pooled_ref[pl.ds(r, 1), :] = pooled_ref[pl.ds(r, 1), :] + s

    @pl.when(i == NBLK - 1)
    def _fc():
        acc = bfc_ref[...]
        for p in range(P):
            acc = acc + jnp.dot(pooled_ref[p:p + 1, :], wf3_ref[p],
                                preferred_element_type=jnp.float32)
        out_ref[...] = acc


@functools.lru_cache(maxsize=None)
def _make_pool_fc():
    return pl.pallas_call(
        _pool_fc_body,
        grid=(NBLK,),
        in_specs=[
            pl.BlockSpec((BN_POOL, CW), lambda i: (i, 0)),
            pl.BlockSpec((P, C, CAT), lambda i: (0, 0, 0)),
            pl.BlockSpec((1, CAT), lambda i: (0, 0)),
        ],
        out_specs=pl.BlockSpec((1, CAT), lambda i: (0, 0)),
        out_shape=jax.ShapeDtypeStruct((1, CAT), jnp.float32),
        scratch_shapes=[pltpu.VMEM((P, C), jnp.float32)],
    )


def _split_weights(W):
    Wr = W.reshape(C, K + 1, C)
    ws = Wr[:, 0, :].T
    wn = Wr[:, 1:, :].transpose(1, 2, 0)
    return ws.astype(jnp.bfloat16), wn.astype(jnp.bfloat16)


def kernel(x, half_edges, W0, b0, W1, b1, W2, b2, Wfc, bfc):
    he = half_edges.astype(jnp.int32)
    idx_full = he.T + (jnp.arange(K, dtype=jnp.int32) * N)[:, None]
    idx_tiled = idx_full.reshape(K, NW * NCH, B).transpose(1, 0, 2)

    sc_conv = _make_sc_conv()

    h = x
    for li, (W, b) in enumerate(((W0, b0), (W1, b1), (W2, b2))):
        first = li == 0
        ws, wn = _split_weights(W)
        zself, zn = _make_project(first, N, 3200)(h, ws, wn, b.reshape(1, C))
        znf = zn.reshape(K * N, C)
        h = sc_conv(zself, znf, idx_tiled)

    wf3 = Wfc.reshape(CAT, P, C).transpose(1, 2, 0) * (1.0 / (N // P))
    out = _make_pool_fc()(h, wf3, bfc.reshape(1, CAT))
    return out.reshape(CAT)


# speedup vs baseline: 1.0168x; 1.0120x over previous
import functools

import jax
import jax.numpy as jnp
from jax import lax
from jax.experimental import pallas as pl
from jax.experimental.pallas import tpu as pltpu
from jax.experimental.pallas import tpu_sc as plsc

N = 320000
C = 128
K = 4
P = 16
CAT = 32

NC = 2
NS = 16
NW = NC * NS
ROWS_W = N // NW
B = 80
NCH = ROWS_W // B
LANES = 16


CW = C // 2


def _pack_row(z):
    zb = z.astype(jnp.bfloat16)
    hi = lax.bitcast_convert_type(zb[:, :CW], jnp.uint16).astype(jnp.uint32)
    lo = lax.bitcast_convert_type(zb[:, CW:], jnp.uint16).astype(jnp.uint32)
    return (hi << 16) | lo


def _unpack_row(v):
    hi = lax.bitcast_convert_type(
        lax.shift_right_logical(v, jnp.uint32(16)).astype(jnp.uint16),
        jnp.bfloat16)
    lo = lax.bitcast_convert_type(
        (v & jnp.uint32(0xFFFF)).astype(jnp.uint16), jnp.bfloat16)
    return hi, lo


def _proj_body(first, x_ref, ws_ref, wn_ref, b_ref, zself_ref, zn_ref):
    if first:
        xa = x_ref[...].astype(jnp.bfloat16)
    else:
        hi, lo = _unpack_row(x_ref[...])
        xa = jnp.concatenate([hi, lo], axis=1)
    dots = [jnp.dot(xa, ws_ref[...], preferred_element_type=jnp.float32)]
    dots += [jnp.dot(xa, wn_ref[k], preferred_element_type=jnp.float32)
             for k in range(K)]
    zself_ref[...] = _pack_row(dots[0] + b_ref[...])
    for k in range(K):
        zn_ref[k] = dots[k + 1]


@functools.lru_cache(maxsize=None)
def _make_project(first, n, bn):
    grid = n // bn
    if first:
        x_spec = pl.BlockSpec((bn, C), lambda i: (i, 0))
    else:
        x_spec = pl.BlockSpec((bn, CW), lambda i: (i, 0))
    ws_spec = pl.BlockSpec((C, C), lambda i: (0, 0))
    wn_spec = pl.BlockSpec((K, C, C), lambda i: (0, 0, 0))
    return pl.pallas_call(
        functools.partial(_proj_body, first),
        grid=(grid,),
        in_specs=[
            x_spec,
            ws_spec,
            wn_spec,
            pl.BlockSpec((1, C), lambda i: (0, 0)),
        ],
        out_specs=[
            pl.BlockSpec((bn, CW), lambda i: (i, 0)),
            pl.BlockSpec((K, bn, C), lambda i: (0, i, 0)),
        ],
        out_shape=[
            jax.ShapeDtypeStruct((n, CW), jnp.uint32),
            jax.ShapeDtypeStruct((K, n, C), jnp.float32),
        ],
    )


_M_HI = jnp.uint32(0xFFFF0000)
_R_HALF = jnp.uint32(0x8000)
_S16 = jnp.uint32(16)


def _sc_combine_rows(acc_v, gbuf_v):
    def f32hi(u):
        return lax.bitcast_convert_type(u & _M_HI, jnp.float32)

    def f32lo(u):
        return lax.bitcast_convert_type(lax.shift_left(u, _S16), jnp.float32)

    def row(r, carry):
        for cc in range(CW // LANES):
            slp = pl.ds(cc * LANES, LANES)
            slo = pl.ds(CW + cc * LANES, LANES)
            u = acc_v[r, slp]
            ga = ((gbuf_v[0, r, slp] + gbuf_v[1, r, slp])
                  + (gbuf_v[2, r, slp] + gbuf_v[3, r, slp]))
            gb = ((gbuf_v[0, r, slo] + gbuf_v[1, r, slo])
                  + (gbuf_v[2, r, slo] + gbuf_v[3, r, slo]))
            a = jnp.maximum(f32hi(u) + ga, 0.0)
            b = jnp.maximum(f32lo(u) + gb, 0.0)
            ua = (lax.bitcast_convert_type(a, jnp.uint32) + _R_HALF) & _M_HI
            ub = lax.shift_right_logical(
                lax.bitcast_convert_type(b, jnp.uint32) + _R_HALF, _S16)
            acc_v[r, slp] = ua | ub
        return carry

    lax.fori_loop(0, B, row, 0)


def _sc_body(zself_hbm, znf_hbm, idx_hbm, out_hbm,
             idx0, idx1, acc0, acc1, gb0, gb1, sg0, sg1, si0, si1):
    wid = lax.axis_index("s") * NC + lax.axis_index("c")
    base_c = wid * NCH
    idxb, accb, gbb = (idx0, idx1), (acc0, acc1), (gb0, gb1)
    sgb, sib = (sg0, sg1), (si0, si1)

    def issue(ci, b):
        for k in range(K):
            pltpu.async_copy(znf_hbm.at[idxb[b].at[k]], gbb[b].at[k], sgb[b])
        pltpu.async_copy(
            zself_hbm.at[pl.ds((base_c + ci) * B, B)], accb[b], sgb[b])

    def drain(b):
        for k in range(K):
            pltpu.make_async_copy(
                znf_hbm.at[idxb[b].at[k]], gbb[b].at[k], sgb[b]).wait()
        pltpu.make_async_copy(
            zself_hbm.at[pl.ds(0, B)], accb[b], sgb[b]).wait()

    def issue_idx(ci, b):
        pltpu.async_copy(idx_hbm.at[base_c + ci], idxb[b], sib[b])

    def drain_idx(b):
        pltpu.make_async_copy(idx_hbm.at[0], idxb[b], sib[b]).wait()

    def process(ci, b):
        drain(b)

        @pl.when(ci + 1 < NCH)
        def _start_next():
            drain_idx(1 - b)
            issue(ci + 1, 1 - b)

        @pl.when(ci + 2 < NCH)
        def _prefetch_idx():
            issue_idx(ci + 2, b)

        _sc_combine_rows(accb[b], gbb[b])
        pltpu.sync_copy(accb[b], out_hbm.at[pl.ds((base_c + ci) * B, B)])

    pltpu.sync_copy(idx_hbm.at[base_c], idxb[0])
    issue(0, 0)
    issue_idx(1, 1)

    def pair(i, carry):
        ci = i * 2
        process(ci, 0)

        @pl.when(ci + 1 < NCH)
        def _odd():
            process(ci + 1, 1)

        return carry

    lax.fori_loop(0, (NCH + 1) // 2, pair, 0)


@functools.lru_cache(maxsize=None)
def _make_sc_conv():
    mesh = plsc.VectorSubcoreMesh(core_axis_name="c", subcore_axis_name="s",
                                  num_cores=NC, num_subcores=NS)
    scratch = [
        pltpu.VMEM((K, B), jnp.int32),
        pltpu.VMEM((K, B), jnp.int32),
        pltpu.VMEM((B, CW), jnp.uint32),
        pltpu.VMEM((B, CW), jnp.uint32),
        pltpu.VMEM((K, B, C), jnp.float32),
        pltpu.VMEM((K, B, C), jnp.float32),
        pltpu.SemaphoreType.DMA,
        pltpu.SemaphoreType.DMA,
        pltpu.SemaphoreType.DMA,
        pltpu.SemaphoreType.DMA,
    ]
    return pl.kernel(
        _sc_body,
        out_type=jax.ShapeDtypeStruct((N, CW), jnp.uint32),
        mesh=mesh,
        scratch_types=scratch,
    )


BN_POOL = 4000
NBLK = N // BN_POOL
BLK_PER_BIN = (N // P) // BN_POOL


def _pool_fc_body(h_ref, wf3_ref, bfc_ref, out_ref, pooled_ref):
    i = pl.program_id(0)
    r = i // BLK_PER_BIN
    ha, hb = _unpack_row(h_ref[...])
    s = jnp.concatenate(
        [jnp.sum(ha.astype(jnp.float32), axis=0, keepdims=True),
         jnp.sum(hb.astype(jnp.float32), axis=0, keepdims=True)], axis=1)

    @pl.when(i % BLK_PER_BIN == 0)
    def _init():
        pooled_ref[pl.ds(r, 1), :] = s

    @pl.when(i % BLK_PER_BIN != 0)
    def _acc():
        pooled_ref[pl.ds(r, 1), :] = pooled_ref[pl.ds(r, 1), :] + s

    @pl.when(i == NBLK - 1)
    def _fc():
        acc = bfc_ref[...]
        for p in range(P):
            acc = acc + jnp.dot(pooled_ref[p:p + 1, :], wf3_ref[p],
                                preferred_element_type=jnp.float32)
        out_ref[...] = acc


@functools.lru_cache(maxsize=None)
def _make_pool_fc():
    return pl.pallas_call(
        _pool_fc_body,
        grid=(NBLK,),
        in_specs=[
            pl.BlockSpec((BN_POOL, CW), lambda i: (i, 0)),
            pl.BlockSpec((P, C, CAT), lambda i: (0, 0, 0)),
            pl.BlockSpec((1, CAT), lambda i: (0, 0)),
        ],
        out_specs=pl.BlockSpec((1, CAT), lambda i: (0, 0)),
        out_shape=jax.ShapeDtypeStruct((1, CAT), jnp.float32),
        scratch_shapes=[pltpu.VMEM((P, C), jnp.float32)],
    )


def _split_weights(W):
    Wr = W.reshape(C, K + 1, C)
    ws = Wr[:, 0, :].T
    wn = Wr[:, 1:, :].transpose(1, 2, 0)
    return ws.astype(jnp.bfloat16), wn.astype(jnp.bfloat16)


def kernel(x, half_edges, W0, b0, W1, b1, W2, b2, Wfc, bfc):
    he = half_edges.astype(jnp.int32)
    idx_full = he.T + (jnp.arange(K, dtype=jnp.int32) * N)[:, None]
    idx_tiled = idx_full.reshape(K, NW * NCH, B).transpose(1, 0, 2)

    sc_conv = _make_sc_conv()

    h = x
    for li, (W, b) in enumerate(((W0, b0), (W1, b1), (W2, b2))):
        first = li == 0
        ws, wn = _split_weights(W)
        zself, zn = _make_project(first, N, 8000)(h, ws, wn, b.reshape(1, C))
        znf = zn.reshape(K * N, C)
        h = sc_conv(zself, znf, idx_tiled)

    wf3 = Wfc.reshape(CAT, P, C).transpose(1, 2, 0) * (1.0 / (N // P))
    out = _make_pool_fc()(h, wf3, bfc.reshape(1, CAT))
    return out.reshape(CAT)
